# Initial kernel scaffold; baseline (speedup 1.0000x reference)
import jax
import jax.numpy as jnp
from jax.experimental import pallas as pl


def kernel(in_attributes, point_cloud):
    raise NotImplementedError("write your pallas kernel here")


# speedup vs baseline: 2.9222x; 2.9222x over previous
import functools

import jax
import jax.numpy as jnp
from jax.experimental import pallas as pl

N = 4096
C = 64
NUM_BINS = 16
BI = 128


def _tc_body(xyz_ref, pci_ref, attr_ref, out_ref):
    i0 = pl.program_id(0) * BI
    xyz = xyz_ref[...]
    xj = xyz[0, :][None, :]
    yj = xyz[1, :][None, :]
    zj = xyz[2, :][None, :]
    pci = pci_ref[...]
    dx = xj - pci[:, 0:1]
    dy = yj - pci[:, 1:2]
    dz = zj - pci[:, 2:3]
    d2 = dx * dx + dy * dy + dz * dz + 1e-12
    dist = jnp.sqrt(d2)
    shell = (dist >= 0.5).astype(jnp.int32)
    octant = (4 * (dx > 0).astype(jnp.int32)
              + 2 * (dy > 0).astype(jnp.int32)
              + (dz > 0).astype(jnp.int32))
    bin_idx = shell * 8 + octant
    jidx = jax.lax.broadcasted_iota(jnp.int32, (BI, N), 1)
    iidx = jax.lax.broadcasted_iota(jnp.int32, (BI, N), 0) + i0
    valid = (dist <= 1.0) & (jidx != iidx)
    bin_eff = jnp.where(valid, bin_idx, NUM_BINS)
    attr = attr_ref[...]
    for b in range(NUM_BINS):
        mask = (bin_eff == b).astype(jnp.float32)
        out_ref[:, b, :] = jnp.dot(mask, attr,
                                   preferred_element_type=jnp.float32)


@jax.jit
def kernel(in_attributes, point_cloud):
    xyz = point_cloud.T.reshape(3, N)
    pci = jnp.pad(point_cloud, ((0, 0), (0, 5)))
    grid = (N // BI,)
    out = pl.pallas_call(
        _tc_body,
        grid=grid,
        in_specs=[
            pl.BlockSpec((3, N), lambda i: (0, 0)),
            pl.BlockSpec((BI, 8), lambda i: (i, 0)),
            pl.BlockSpec((N, C), lambda i: (0, 0)),
        ],
        out_specs=pl.BlockSpec((BI, NUM_BINS, C), lambda i: (i, 0, 0)),
        out_shape=jax.ShapeDtypeStruct((N, NUM_BINS, C), jnp.float32),
    )(xyz, pci, in_attributes)
    return out
